# baseline (speedup 1.0000x reference)
import functools

import jax
import jax.numpy as jnp
from jax import lax
from jax.experimental import pallas as pl
from jax.experimental.pallas import tpu as pltpu
from jax.experimental.pallas import tpu_sc as plsc

_NUM_CORES = 2
_NUM_SUBCORES = 16
_BANDS = 10
_ACTIVE = 3 * _BANDS


@jax.jit
def _sc_copy(x):
    _, c, h, w = x.shape
    rows = h // _BANDS

    def body(x_hbm, out_hbm, buf, in_sem, out_sem):
        wid = lax.axis_index("s") * _NUM_CORES + lax.axis_index("c")

        @pl.when(wid < _ACTIVE)
        def _():
            ch = wid // _BANDS
            r0 = (wid % _BANDS) * rows
            pltpu.async_copy(
                x_hbm.at[0, ch, pl.ds(r0, rows), :], buf, in_sem).wait()
            pltpu.async_copy(
                buf, out_hbm.at[0, ch, pl.ds(r0, rows), :], out_sem).wait()

    mesh = plsc.VectorSubcoreMesh(core_axis_name="c", subcore_axis_name="s")
    return pl.kernel(
        body,
        out_type=jax.ShapeDtypeStruct(x.shape, x.dtype),
        mesh=mesh,
        scratch_types=[
            pltpu.VMEM((rows, w), jnp.float32),
            pltpu.SemaphoreType.DMA,
            pltpu.SemaphoreType.DMA,
        ],
    )(x)


def kernel(x, buffer):
    return _sc_copy(x)
